# baseline (speedup 1.0000x reference)
import functools

import jax
import jax.numpy as jnp
from jax import lax
from jax.experimental import pallas as pl
from jax.experimental.pallas import tpu as pltpu
from jax.experimental.pallas import tpu_sc as plsc

_B = 16384
_D = 64
_L = 16
_NC = 2
_NS = 16
_NW = _NC * _NS
_BPW = _B // _NW
_PACK = 4
_DP = 128
_DW = _D // 2
_CHUNK = 128
_NCHUNK = _BPW // _CHUNK


@functools.partial(
    pl.kernel,
    mesh=plsc.VectorSubcoreMesh(core_axis_name="c", subcore_axis_name="s"),
    out_type=jax.ShapeDtypeStruct((_B,), jnp.float32),
    compiler_params=pltpu.CompilerParams(
        use_tc_tiling_on_sc=True, needs_layout_passes=False),
    scratch_types=[
        pltpu.VMEM((_NCHUNK, _CHUNK), jnp.int32),
        pltpu.VMEM((_NCHUNK, _CHUNK), jnp.int32),
        pltpu.VMEM((_NCHUNK, _CHUNK), jnp.int32),
        pltpu.VMEM((_NCHUNK, _CHUNK), jnp.int32),
        pltpu.VMEM((_CHUNK, _DP), jnp.float32),
        pltpu.VMEM((_CHUNK, _DP), jnp.float32),
        pltpu.VMEM((_CHUNK, _DP), jnp.float32),
        pltpu.VMEM((_CHUNK, _DP), jnp.float32),
        pltpu.VMEM((_D,), jnp.float32),
        pltpu.VMEM((_L,), jnp.float32),
        pltpu.VMEM((_BPW,), jnp.float32),
        pltpu.SemaphoreType.DMA,
        pltpu.SemaphoreType.DMA,
    ],
)
def _gmf_sc(user_hbm, item_hbm, iu_hbm, iv_hbm, w_hbm, bias_hbm, out_hbm,
            iu_v, iv_v, iu4_v, iv4_v, ub0, ub1, vb0, vb1, w_v, bias_v, out_v,
            sem0, sem1):
    wid = lax.axis_index("s") * _NC + lax.axis_index("c")
    base = wid * _BPW
    ubufs = (ub0, ub1)
    vbufs = (vb0, vb1)
    sems = (sem0, sem1)

    pltpu.sync_copy(iu_hbm.at[pl.ds(wid * _NCHUNK, _NCHUNK)], iu_v)
    pltpu.sync_copy(iv_hbm.at[pl.ds(wid * _NCHUNK, _NCHUNK)], iv_v)
    pltpu.sync_copy(w_hbm, w_v)
    pltpu.sync_copy(bias_hbm, bias_v)

    for j in range(_NCHUNK):
        for t in range(_CHUNK // _L):
            s = pl.ds(t * _L, _L)
            iu4_v[j, s] = lax.shift_right_logical(iu_v[j, s], 2)
            iv4_v[j, s] = lax.shift_right_logical(iv_v[j, s], 2)

    def fire(j):
        k = j % 2
        return (pltpu.async_copy(user_hbm.at[iu4_v.at[j]], ubufs[k], sems[k]),
                pltpu.async_copy(item_hbm.at[iv4_v.at[j]], vbufs[k], sems[k]))

    w0 = w_v[pl.ds(0 * _L, _L)]
    w1 = w_v[pl.ds(1 * _L, _L)]
    w2 = w_v[pl.ds(2 * _L, _L)]
    w3 = w_v[pl.ds(3 * _L, _L)]

    lane = lax.iota(jnp.int32, _L)
    lo_half = lane < (_L // 2)
    perm_even = (lane * 2) % _L
    perm_odd = perm_even + 1

    def shuf(x, perm):
        return lax.gather(
            x, perm[:, None],
            lax.GatherDimensionNumbers(
                offset_dims=(), collapsed_slice_dims=(0,), start_index_map=(0,)),
            slice_sizes=(1,),
            mode=lax.GatherScatterMode.PROMISE_IN_BOUNDS)

    def hadd(a, b):
        return jnp.where(lo_half,
                         shuf(a, perm_even) + shuf(a, perm_odd),
                         shuf(b, perm_even) + shuf(b, perm_odd))

    def unpack2(x16):
        b32 = plsc.bitcast(x16, jnp.bfloat16)
        return plsc.unpack(b32, format=plsc.PackFormat.INTERLEAVED)

    cps = {0: fire(0)}
    for j in range(_NCHUNK):
        if j + 1 < _NCHUNK:
            cps[j + 1] = fire(j + 1)
        for c in cps.pop(j):
            c.wait()
        u_rows = ubufs[j % 2]
        v_rows = vbufs[j % 2]

        def block_body(blk, carry, u_rows=u_rows, v_rows=v_rows, j=j):
            base_r = blk * _L
            qu = jnp.bitwise_and(iu_v[j, pl.ds(base_r, _L)], _PACK - 1) * _DW
            qv = jnp.bitwise_and(iv_v[j, pl.ds(base_r, _L)], _PACK - 1) * _DW
            ps = []
            for k in range(_L):
                r = base_r + k
                bcast_k = lane * 0 + k
                qu_k = shuf(qu, bcast_k)
                qv_k = shuf(qv, bcast_k)
                row_id = lane * 0 + r
                pu0 = plsc.load_gather(u_rows, [row_id, qu_k + lane])
                pu1 = plsc.load_gather(u_rows, [row_id, qu_k + (lane + _L)])
                pv0 = plsc.load_gather(v_rows, [row_id, qv_k + lane])
                pv1 = plsc.load_gather(v_rows, [row_id, qv_k + (lane + _L)])
                ue0, uo0 = unpack2(pu0)
                ue1, uo1 = unpack2(pu1)
                ve0, vo0 = unpack2(pv0)
                ve1, vo1 = unpack2(pv1)
                p = (ue0 * w0) * ve0 + (uo0 * w1) * vo0
                p = p + (ue1 * w2) * ve1 + (uo1 * w3) * vo1
                ps.append(p)
            while len(ps) > 1:
                ps = [hadd(ps[i], ps[i + 1]) for i in range(0, len(ps), 2)]
            out_v[pl.ds(j * _CHUNK + base_r, _L)] = ps[0]
            return carry

        lax.fori_loop(0, _CHUNK // _L, block_body, 0)

    bv = bias_v[...]
    for i in range(_BPW // _L):
        x = out_v[pl.ds(i * _L, _L)] + bv
        out_v[pl.ds(i * _L, _L)] = 1.0 / (1.0 + jnp.exp(-x))

    pltpu.sync_copy(out_v, out_hbm.at[pl.ds(base, _BPW)])


def _pack_table(t):
    tb = t.astype(jnp.bfloat16).reshape(t.shape[0], _DW, 2)
    tp = lax.bitcast_convert_type(tb, jnp.float32)
    return tp.reshape(t.shape[0] // _PACK, _DP)


def kernel(inputs, user_table, item_table, W, b):
    idx = inputs.astype(jnp.int32)
    iu = idx[:, 0].reshape(_NW * _NCHUNK, _CHUNK)
    iv = idx[:, 1].reshape(_NW * _NCHUNK, _CHUNK)
    up = _pack_table(user_table)
    vp = _pack_table(item_table)
    w = W.reshape(_D).astype(jnp.float32)
    wsplit = jnp.concatenate(
        [w[0:32:2], w[1:32:2], w[32:64:2], w[33:64:2]])
    bias = jnp.broadcast_to(b.astype(jnp.float32), (_L,))
    out = _gmf_sc(up, vp, iu, iv, wsplit, bias)
    return out.reshape(_B, 1)


# speedup vs baseline: 1.5973x; 1.5973x over previous
import functools

import jax
import jax.numpy as jnp
from jax import lax
from jax.experimental import pallas as pl
from jax.experimental.pallas import tpu as pltpu
from jax.experimental.pallas import tpu_sc as plsc

_B = 16384
_D = 64
_L = 16
_NC = 2
_NS = 16
_NW = _NC * _NS
_BPW = _B // _NW
_PACK = 4
_DP = 128
_DW = _D // 2
_CHUNK = 128
_NCHUNK = _BPW // _CHUNK


@functools.partial(
    pl.kernel,
    mesh=plsc.VectorSubcoreMesh(core_axis_name="c", subcore_axis_name="s"),
    out_type=jax.ShapeDtypeStruct((_B,), jnp.float32),
    compiler_params=pltpu.CompilerParams(
        use_tc_tiling_on_sc=True, needs_layout_passes=False),
    scratch_types=[
        pltpu.VMEM((_NCHUNK, _CHUNK), jnp.int32),
        pltpu.VMEM((_NCHUNK, _CHUNK), jnp.int32),
        pltpu.VMEM((_NCHUNK, _CHUNK), jnp.int32),
        pltpu.VMEM((_NCHUNK, _CHUNK), jnp.int32),
        pltpu.VMEM((_CHUNK, _DP), jnp.float32),
        pltpu.VMEM((_CHUNK, _DP), jnp.float32),
        pltpu.VMEM((_CHUNK, _DP), jnp.float32),
        pltpu.VMEM((_CHUNK, _DP), jnp.float32),
        pltpu.VMEM((_D,), jnp.float32),
        pltpu.VMEM((_L,), jnp.float32),
        pltpu.VMEM((_BPW,), jnp.float32),
        pltpu.SemaphoreType.DMA,
        pltpu.SemaphoreType.DMA,
    ],
)
def _gmf_sc(user_hbm, item_hbm, iu_hbm, iv_hbm, w_hbm, bias_hbm, out_hbm,
            iu_v, iv_v, iu4_v, iv4_v, ub0, ub1, vb0, vb1, w_v, bias_v, out_v,
            sem0, sem1):
    wid = lax.axis_index("s") * _NC + lax.axis_index("c")
    base = wid * _BPW
    ubufs = (ub0, ub1)
    vbufs = (vb0, vb1)
    sems = (sem0, sem1)

    pltpu.sync_copy(iu_hbm.at[pl.ds(wid * _NCHUNK, _NCHUNK)], iu_v)
    pltpu.sync_copy(iv_hbm.at[pl.ds(wid * _NCHUNK, _NCHUNK)], iv_v)
    pltpu.sync_copy(w_hbm, w_v)
    pltpu.sync_copy(bias_hbm, bias_v)

    for j in range(_NCHUNK):
        for t in range(_CHUNK // _L):
            s = pl.ds(t * _L, _L)
            iu4_v[j, s] = lax.shift_right_logical(iu_v[j, s], 2)
            iv4_v[j, s] = lax.shift_right_logical(iv_v[j, s], 2)

    def fire(j):
        k = j % 2
        return (pltpu.async_copy(user_hbm.at[iu4_v.at[j]], ubufs[k], sems[k]),
                pltpu.async_copy(item_hbm.at[iv4_v.at[j]], vbufs[k], sems[k]))

    w0 = w_v[pl.ds(0 * _L, _L)]
    w1 = w_v[pl.ds(1 * _L, _L)]
    w2 = w_v[pl.ds(2 * _L, _L)]
    w3 = w_v[pl.ds(3 * _L, _L)]

    lane = lax.iota(jnp.int32, _L)
    lo_half = lane < (_L // 2)
    perm_even = (lane * 2) % _L
    perm_odd = perm_even + 1

    def shuf(x, perm):
        return lax.gather(
            x, perm[:, None],
            lax.GatherDimensionNumbers(
                offset_dims=(), collapsed_slice_dims=(0,), start_index_map=(0,)),
            slice_sizes=(1,),
            mode=lax.GatherScatterMode.PROMISE_IN_BOUNDS)

    def hadd(a, b):
        return jnp.where(lo_half,
                         shuf(a, perm_even) + shuf(a, perm_odd),
                         shuf(b, perm_even) + shuf(b, perm_odd))

    def unpack2(x16):
        b32 = plsc.bitcast(x16, jnp.bfloat16)
        return plsc.unpack(b32, format=plsc.PackFormat.INTERLEAVED)

    cps = {0: fire(0)}
    for j in range(_NCHUNK):
        if j + 1 < _NCHUNK:
            cps[j + 1] = fire(j + 1)
        for c in cps.pop(j):
            c.wait()
        u_rows = ubufs[j % 2]
        v_rows = vbufs[j % 2]

        def block_body(blk, carry, u_rows=u_rows, v_rows=v_rows, j=j):
            base_r = blk * _L
            qu = jnp.bitwise_and(iu_v[j, pl.ds(base_r, _L)], _PACK - 1) * _DW
            qv = jnp.bitwise_and(iv_v[j, pl.ds(base_r, _L)], _PACK - 1) * _DW
            ps = []
            for k in range(_L):
                r = base_r + k
                bcast_k = lane * 0 + k
                qu_k = shuf(qu, bcast_k)
                qv_k = shuf(qv, bcast_k)
                row_id = lane * 0 + r
                pu0 = plsc.load_gather(u_rows, [row_id, qu_k + lane])
                pu1 = plsc.load_gather(u_rows, [row_id, qu_k + (lane + _L)])
                pv0 = plsc.load_gather(v_rows, [row_id, qv_k + lane])
                pv1 = plsc.load_gather(v_rows, [row_id, qv_k + (lane + _L)])
                ue0, uo0 = unpack2(pu0)
                ue1, uo1 = unpack2(pu1)
                ve0, vo0 = unpack2(pv0)
                ve1, vo1 = unpack2(pv1)
                p = (ue0 * w0) * ve0 + (uo0 * w1) * vo0
                p = p + (ue1 * w2) * ve1 + (uo1 * w3) * vo1
                ps.append(p)
            while len(ps) > 1:
                ps = [hadd(ps[i], ps[i + 1]) for i in range(0, len(ps), 2)]
            out_v[pl.ds(j * _CHUNK + base_r, _L)] = ps[0]
            return carry

        lax.fori_loop(0, _CHUNK // _L, block_body, 0)

    bv = bias_v[...]
    for i in range(_BPW // _L):
        x = out_v[pl.ds(i * _L, _L)] + bv
        out_v[pl.ds(i * _L, _L)] = 1.0 / (1.0 + jnp.exp(-x))

    pltpu.sync_copy(out_v, out_hbm.at[pl.ds(base, _BPW)])


_V = 100000
_VP = _V // _PACK
_NITEM = 782
_IPW = 25
_TAIL = _V - 781 * _CHUNK


@functools.partial(
    pl.kernel,
    mesh=plsc.VectorSubcoreMesh(core_axis_name="c", subcore_axis_name="s"),
    out_type=(jax.ShapeDtypeStruct((_VP, _DP), jnp.float32),
              jax.ShapeDtypeStruct((_VP, _DP), jnp.float32)),
    compiler_params=pltpu.CompilerParams(
        use_tc_tiling_on_sc=True, needs_layout_passes=False),
    scratch_types=[
        pltpu.VMEM((_D, _CHUNK), jnp.float32),
        pltpu.VMEM((_D, _CHUNK), jnp.float32),
        pltpu.VMEM((_CHUNK // _PACK, _DP), jnp.float32),
        pltpu.VMEM((_CHUNK // _PACK, _DP), jnp.float32),
        pltpu.VMEM((_D, _TAIL), jnp.float32),
        pltpu.VMEM((_TAIL // _PACK, _DP), jnp.float32),
        pltpu.SemaphoreType.DMA,
        pltpu.SemaphoreType.DMA,
        pltpu.SemaphoreType.DMA,
        pltpu.SemaphoreType.DMA,
    ],
)
def _pack_sc(ut_hbm, vt_hbm, up_hbm, vp_hbm,
             inb0, inb1, outb0, outb1, in_t, out_t,
             semi0, semi1, semo0, semo1):
    wid = lax.axis_index("s") * _NC + lax.axis_index("c")
    inb = (inb0, inb1)
    outb = (outb0, outb1)
    semi = (semi0, semi1)
    semo = (semo0, semo1)

    lane = lax.iota(jnp.int32, _L)
    fe = (lane * 2, lane * 2 + 32)
    fo = (fe[0] + 1, fe[1] + 1)

    def pack_rows(src_v, dst_v, nrows_p):
        def p_body(p, carry):
            for m in range(_PACK):
                col = lane * 0 + (p * _PACK + m)
                for c in range(2):
                    a = plsc.load_gather(src_v, [fe[c], col])
                    b = plsc.load_gather(src_v, [fo[c], col])
                    w16 = plsc.bitcast(
                        plsc.pack(a, b, format=plsc.PackFormat.INTERLEAVED),
                        jnp.float32)
                    dst_v[p, pl.ds(m * 32 + c * _L, _L)] = w16
            return carry
        lax.fori_loop(0, nrows_p, p_body, 0)

    for src, dst in ((ut_hbm, up_hbm), (vt_hbm, vp_hbm)):
        def item_of(j):
            return wid * _IPW + j

        def fire_in(j, k):
            t = item_of(j)

            @pl.when(t < _NITEM - 1)
            def _():
                pltpu.async_copy(
                    src.at[:, pl.ds(t * _CHUNK, _CHUNK)], inb[k], semi[k])

        def wait_out(j):
            t2 = item_of(j)

            @pl.when(t2 < _NITEM - 1)
            def _():
                pltpu.make_async_copy(
                    outb[j % 2], dst.at[pl.ds(t2 * (_CHUNK // _PACK),
                                              _CHUNK // _PACK)],
                    semo[j % 2]).wait()

        def handle(j, k):
            t = item_of(j)
            if j + 1 < _IPW:
                fire_in(j + 1, 1 - k)

            @pl.when(t < _NITEM - 1)
            def _():
                pltpu.make_async_copy(
                    src.at[:, pl.ds(t * _CHUNK, _CHUNK)], inb[k], semi[k]).wait()

            if j >= 2:
                wait_out(j - 2)

            @pl.when(t < _NITEM - 1)
            def _():
                pack_rows(inb[k], outb[k], _CHUNK // _PACK)
                pltpu.async_copy(
                    outb[k], dst.at[pl.ds(t * (_CHUNK // _PACK),
                                          _CHUNK // _PACK)], semo[k])

            if (_NITEM - 1 - j) % _IPW == 0:
                @pl.when(t == _NITEM - 1)
                def _():
                    pltpu.sync_copy(src.at[:, pl.ds(781 * _CHUNK, _TAIL)], in_t)
                    pack_rows(in_t, out_t, _TAIL // _PACK)
                    pltpu.sync_copy(
                        out_t, dst.at[pl.ds(781 * (_CHUNK // _PACK),
                                            _TAIL // _PACK)])

        fire_in(0, 0)
        for j in range(_IPW):
            handle(j, j % 2)
        wait_out(_IPW - 2)
        wait_out(_IPW - 1)


def kernel(inputs, user_table, item_table, W, b):
    idx = inputs.astype(jnp.int32)
    iu = idx[:, 0].reshape(_NW * _NCHUNK, _CHUNK)
    iv = idx[:, 1].reshape(_NW * _NCHUNK, _CHUNK)
    up, vp = _pack_sc(jnp.transpose(user_table), jnp.transpose(item_table))
    w = W.reshape(_D).astype(jnp.float32)
    wsplit = jnp.concatenate(
        [w[0:32:2], w[1:32:2], w[32:64:2], w[33:64:2]])
    bias = jnp.broadcast_to(b.astype(jnp.float32), (_L,))
    out = _gmf_sc(up, vp, iu, iv, wsplit, bias)
    return out.reshape(_B, 1)


# speedup vs baseline: 3.6827x; 2.3056x over previous
import functools

import jax
import jax.numpy as jnp
from jax import lax
from jax.experimental import pallas as pl
from jax.experimental.pallas import tpu as pltpu
from jax.experimental.pallas import tpu_sc as plsc

_B = 16384
_D = 64
_L = 16
_NC = 2
_NS = 16
_NW = _NC * _NS
_BPW = _B // _NW
_PACK = 4
_DP = 128
_DW = _D // 2
_CHUNK = 128
_NCHUNK = _BPW // _CHUNK


@functools.partial(
    pl.kernel,
    mesh=plsc.VectorSubcoreMesh(core_axis_name="c", subcore_axis_name="s"),
    out_type=jax.ShapeDtypeStruct((_B,), jnp.float32),
    compiler_params=pltpu.CompilerParams(
        use_tc_tiling_on_sc=True, needs_layout_passes=False),
    scratch_types=[
        pltpu.VMEM((_NCHUNK, _CHUNK), jnp.int32),
        pltpu.VMEM((_NCHUNK, _CHUNK), jnp.int32),
        pltpu.VMEM((_NCHUNK, _CHUNK), jnp.int32),
        pltpu.VMEM((_NCHUNK, _CHUNK), jnp.int32),
        pltpu.VMEM((_CHUNK, _DP), jnp.float32),
        pltpu.VMEM((_CHUNK, _DP), jnp.float32),
        pltpu.VMEM((_CHUNK, _DP), jnp.float32),
        pltpu.VMEM((_CHUNK, _DP), jnp.float32),
        pltpu.VMEM((_D,), jnp.float32),
        pltpu.VMEM((_L,), jnp.float32),
        pltpu.VMEM((_BPW,), jnp.float32),
        pltpu.SemaphoreType.DMA,
        pltpu.SemaphoreType.DMA,
    ],
)
def _gmf_sc(user_hbm, item_hbm, iu_hbm, iv_hbm, w_hbm, bias_hbm, out_hbm,
            iu_v, iv_v, iu4_v, iv4_v, ub0, ub1, vb0, vb1, w_v, bias_v, out_v,
            sem0, sem1):
    wid = lax.axis_index("s") * _NC + lax.axis_index("c")
    base = wid * _BPW
    ubufs = (ub0, ub1)
    vbufs = (vb0, vb1)
    sems = (sem0, sem1)

    pltpu.sync_copy(iu_hbm.at[pl.ds(wid * _NCHUNK, _NCHUNK)], iu_v)
    pltpu.sync_copy(iv_hbm.at[pl.ds(wid * _NCHUNK, _NCHUNK)], iv_v)
    pltpu.sync_copy(w_hbm, w_v)
    pltpu.sync_copy(bias_hbm, bias_v)

    for j in range(_NCHUNK):
        for t in range(_CHUNK // _L):
            s = pl.ds(t * _L, _L)
            iu4_v[j, s] = lax.shift_right_logical(iu_v[j, s], 2)
            iv4_v[j, s] = lax.shift_right_logical(iv_v[j, s], 2)

    def fire(j):
        k = j % 2
        return (pltpu.async_copy(user_hbm.at[iu4_v.at[j]], ubufs[k], sems[k]),
                pltpu.async_copy(item_hbm.at[iv4_v.at[j]], vbufs[k], sems[k]))

    w0 = w_v[pl.ds(0 * _L, _L)]
    w1 = w_v[pl.ds(1 * _L, _L)]
    w2 = w_v[pl.ds(2 * _L, _L)]
    w3 = w_v[pl.ds(3 * _L, _L)]

    lane = lax.iota(jnp.int32, _L)
    lo_half = lane < (_L // 2)
    perm_even = (lane * 2) % _L
    perm_odd = perm_even + 1

    def shuf(x, perm):
        return lax.gather(
            x, perm[:, None],
            lax.GatherDimensionNumbers(
                offset_dims=(), collapsed_slice_dims=(0,), start_index_map=(0,)),
            slice_sizes=(1,),
            mode=lax.GatherScatterMode.PROMISE_IN_BOUNDS)

    def hadd(a, b):
        return jnp.where(lo_half,
                         shuf(a, perm_even) + shuf(a, perm_odd),
                         shuf(b, perm_even) + shuf(b, perm_odd))

    def unpack2(x16):
        b32 = plsc.bitcast(x16, jnp.bfloat16)
        return plsc.unpack(b32, format=plsc.PackFormat.INTERLEAVED)

    cps = {0: fire(0)}
    for j in range(_NCHUNK):
        if j + 1 < _NCHUNK:
            cps[j + 1] = fire(j + 1)
        for c in cps.pop(j):
            c.wait()
        u_rows = ubufs[j % 2]
        v_rows = vbufs[j % 2]

        def block_body(blk, carry, u_rows=u_rows, v_rows=v_rows, j=j):
            base_r = blk * _L
            qu = jnp.bitwise_and(iu_v[j, pl.ds(base_r, _L)], _PACK - 1) * _DW
            qv = jnp.bitwise_and(iv_v[j, pl.ds(base_r, _L)], _PACK - 1) * _DW
            ps = []
            for k in range(_L):
                r = base_r + k
                bcast_k = lane * 0 + k
                qu_k = shuf(qu, bcast_k)
                qv_k = shuf(qv, bcast_k)
                row_id = lane * 0 + r
                pu0 = plsc.load_gather(u_rows, [row_id, qu_k + lane])
                pu1 = plsc.load_gather(u_rows, [row_id, qu_k + (lane + _L)])
                pv0 = plsc.load_gather(v_rows, [row_id, qv_k + lane])
                pv1 = plsc.load_gather(v_rows, [row_id, qv_k + (lane + _L)])
                ue0, uo0 = unpack2(pu0)
                ue1, uo1 = unpack2(pu1)
                ve0, vo0 = unpack2(pv0)
                ve1, vo1 = unpack2(pv1)
                p = (ue0 * w0) * ve0 + (uo0 * w1) * vo0
                p = p + (ue1 * w2) * ve1 + (uo1 * w3) * vo1
                ps.append(p)
            while len(ps) > 1:
                ps = [hadd(ps[i], ps[i + 1]) for i in range(0, len(ps), 2)]
            out_v[pl.ds(j * _CHUNK + base_r, _L)] = ps[0]
            return carry

        lax.fori_loop(0, _CHUNK // _L, block_body, 0)

    bv = bias_v[...]
    for i in range(_BPW // _L):
        x = out_v[pl.ds(i * _L, _L)] + bv
        out_v[pl.ds(i * _L, _L)] = 1.0 / (1.0 + jnp.exp(-x))

    pltpu.sync_copy(out_v, out_hbm.at[pl.ds(base, _BPW)])


_V = 100000
_VP = _V // _PACK
_CBLK = 4096
_GRID = (_V + _CBLK - 1) // _CBLK


def _tc_pack_body(ut_ref, vt_ref, up_ref, vp_ref):
    for src, dst in ((ut_ref, up_ref), (vt_ref, vp_ref)):
        t = jnp.transpose(src[...])
        a = lax.bitcast_convert_type(t[:, 0:32], jnp.uint32)
        b = lax.bitcast_convert_type(t[:, 32:64], jnp.uint32)
        half = jnp.uint32(0x8000)
        hi_mask = jnp.uint32(0xFFFF0000)
        w = ((a + half) >> 16) | ((b + half) & hi_mask)
        w3 = w.reshape(_CBLK // _PACK, _PACK, 32)
        w = jnp.concatenate(
            [w3[:, m, :] for m in range(_PACK)], axis=1)
        dst[...] = lax.bitcast_convert_type(w, jnp.float32)


_tc_pack = pl.pallas_call(
    _tc_pack_body,
    grid=(_GRID,),
    in_specs=[
        pl.BlockSpec((_D, _CBLK), lambda i: (0, i)),
        pl.BlockSpec((_D, _CBLK), lambda i: (0, i)),
    ],
    out_specs=[
        pl.BlockSpec((_CBLK // _PACK, _DP), lambda i: (i, 0)),
        pl.BlockSpec((_CBLK // _PACK, _DP), lambda i: (i, 0)),
    ],
    out_shape=[
        jax.ShapeDtypeStruct((_VP, _DP), jnp.float32),
        jax.ShapeDtypeStruct((_VP, _DP), jnp.float32),
    ],
)


def kernel(inputs, user_table, item_table, W, b):
    idx = inputs.astype(jnp.int32)
    iu = idx[:, 0].reshape(_NW * _NCHUNK, _CHUNK)
    iv = idx[:, 1].reshape(_NW * _NCHUNK, _CHUNK)
    up, vp = _tc_pack(jnp.transpose(user_table), jnp.transpose(item_table))
    w = W.reshape(_D).astype(jnp.float32)
    wsplit = jnp.concatenate(
        [w[0:16], w[32:48], w[16:32], w[48:64]])
    bias = jnp.broadcast_to(b.astype(jnp.float32), (_L,))
    out = _gmf_sc(up, vp, iu, iv, wsplit, bias)
    return out.reshape(_B, 1)


# speedup vs baseline: 4.4430x; 1.2064x over previous
import functools

import jax
import jax.numpy as jnp
from jax import lax
from jax.experimental import pallas as pl
from jax.experimental.pallas import tpu as pltpu
from jax.experimental.pallas import tpu_sc as plsc

_B = 16384
_D = 64
_L = 16
_NC = 2
_NS = 16
_NW = _NC * _NS
_BPW = _B // _NW
_PACK = 4
_DP = 128
_DW = _D // 2
_CHUNK = 128
_NCHUNK = _BPW // _CHUNK


@functools.partial(
    pl.kernel,
    mesh=plsc.VectorSubcoreMesh(core_axis_name="c", subcore_axis_name="s"),
    out_type=jax.ShapeDtypeStruct((_B,), jnp.float32),
    compiler_params=pltpu.CompilerParams(
        use_tc_tiling_on_sc=True, needs_layout_passes=False),
    scratch_types=[
        pltpu.VMEM((_NCHUNK, _CHUNK), jnp.int32),
        pltpu.VMEM((_NCHUNK, _CHUNK), jnp.int32),
        pltpu.VMEM((_NCHUNK, _CHUNK), jnp.int32),
        pltpu.VMEM((_NCHUNK, _CHUNK), jnp.int32),
        pltpu.VMEM((_CHUNK, _DP), jnp.float32),
        pltpu.VMEM((_CHUNK, _DP), jnp.float32),
        pltpu.VMEM((_CHUNK, _DP), jnp.float32),
        pltpu.VMEM((_CHUNK, _DP), jnp.float32),
        pltpu.VMEM((_D,), jnp.float32),
        pltpu.VMEM((_L,), jnp.float32),
        pltpu.VMEM((_BPW,), jnp.float32),
        pltpu.SemaphoreType.DMA,
        pltpu.SemaphoreType.DMA,
    ],
)
def _gmf_sc(user_hbm, item_hbm, iu_hbm, iv_hbm, w_hbm, bias_hbm, out_hbm,
            iu_v, iv_v, iu4_v, iv4_v, ub0, ub1, vb0, vb1, w_v, bias_v, out_v,
            sem0, sem1):
    wid = lax.axis_index("s") * _NC + lax.axis_index("c")
    base = wid * _BPW
    ubufs = (ub0, ub1)
    vbufs = (vb0, vb1)
    sems = (sem0, sem1)

    pltpu.sync_copy(iu_hbm.at[pl.ds(wid * _NCHUNK, _NCHUNK)], iu_v)
    pltpu.sync_copy(iv_hbm.at[pl.ds(wid * _NCHUNK, _NCHUNK)], iv_v)
    pltpu.sync_copy(w_hbm, w_v)
    pltpu.sync_copy(bias_hbm, bias_v)

    for j in range(_NCHUNK):
        for t in range(_CHUNK // _L):
            s = pl.ds(t * _L, _L)
            iu4_v[j, s] = lax.shift_right_logical(iu_v[j, s], 2)
            iv4_v[j, s] = lax.shift_right_logical(iv_v[j, s], 2)

    def fire(j):
        k = j % 2
        return (pltpu.async_copy(user_hbm.at[iu4_v.at[j]], ubufs[k], sems[k]),
                pltpu.async_copy(item_hbm.at[iv4_v.at[j]], vbufs[k], sems[k]))

    w0 = w_v[pl.ds(0 * _L, _L)]
    w1 = w_v[pl.ds(1 * _L, _L)]
    w2 = w_v[pl.ds(2 * _L, _L)]
    w3 = w_v[pl.ds(3 * _L, _L)]

    lane = lax.iota(jnp.int32, _L)
    lo_half = lane < (_L // 2)
    perm_even = (lane * 2) % _L
    perm_odd = perm_even + 1

    def shuf(x, perm):
        return lax.gather(
            x, perm[:, None],
            lax.GatherDimensionNumbers(
                offset_dims=(), collapsed_slice_dims=(0,), start_index_map=(0,)),
            slice_sizes=(1,),
            mode=lax.GatherScatterMode.PROMISE_IN_BOUNDS)

    def hadd(a, b):
        return jnp.where(lo_half,
                         shuf(a, perm_even) + shuf(a, perm_odd),
                         shuf(b, perm_even) + shuf(b, perm_odd))

    def unpack2(x16):
        b32 = plsc.bitcast(x16, jnp.bfloat16)
        return plsc.unpack(b32, format=plsc.PackFormat.INTERLEAVED)

    cps = {0: fire(0)}
    for j in range(_NCHUNK):
        if j + 1 < _NCHUNK:
            cps[j + 1] = fire(j + 1)
        for c in cps.pop(j):
            c.wait()
        u_rows = ubufs[j % 2]
        v_rows = vbufs[j % 2]

        def block_body(blk, carry, u_rows=u_rows, v_rows=v_rows, j=j):
            base_r = blk * _L
            qu = jnp.bitwise_and(iu_v[j, pl.ds(base_r, _L)], _PACK - 1) * _DW
            qv = jnp.bitwise_and(iv_v[j, pl.ds(base_r, _L)], _PACK - 1) * _DW
            ps = []
            for k in range(_L):
                r = base_r + k
                bcast_k = lane * 0 + k
                qu_k = shuf(qu, bcast_k)
                qv_k = shuf(qv, bcast_k)
                row_id = lane * 0 + r
                pu0 = plsc.load_gather(u_rows, [row_id, qu_k + lane])
                pu1 = plsc.load_gather(u_rows, [row_id, qu_k + (lane + _L)])
                pv0 = plsc.load_gather(v_rows, [row_id, qv_k + lane])
                pv1 = plsc.load_gather(v_rows, [row_id, qv_k + (lane + _L)])
                ue0, uo0 = unpack2(pu0)
                ue1, uo1 = unpack2(pu1)
                ve0, vo0 = unpack2(pv0)
                ve1, vo1 = unpack2(pv1)
                p = (ue0 * w0) * ve0 + (uo0 * w1) * vo0
                p = p + (ue1 * w2) * ve1 + (uo1 * w3) * vo1
                ps.append(p)
            while len(ps) > 1:
                ps = [hadd(ps[i], ps[i + 1]) for i in range(0, len(ps), 2)]
            out_v[pl.ds(j * _CHUNK + base_r, _L)] = ps[0]
            return carry

        lax.fori_loop(0, _CHUNK // _L, block_body, 0)

    bv = bias_v[...]
    for i in range(_BPW // _L):
        x = out_v[pl.ds(i * _L, _L)] + bv
        out_v[pl.ds(i * _L, _L)] = 1.0 / (1.0 + jnp.exp(-x))

    pltpu.sync_copy(out_v, out_hbm.at[pl.ds(base, _BPW)])


_V = 100000
_VP = _V // _PACK
_CBLK = 4096
_GRID = (_V + _CBLK - 1) // _CBLK


def _tc_pack_body(ut_ref, vt_ref, up_ref, vp_ref):
    for src, dst in ((ut_ref, up_ref), (vt_ref, vp_ref)):
        x = lax.bitcast_convert_type(src[...], jnp.uint32)
        half = jnp.uint32(0x8000)
        hi_mask = jnp.uint32(0xFFFF0000)
        w = ((x[0:32, :] + half) >> 16) | ((x[32:64, :] + half) & hi_mask)
        t = jnp.transpose(w)
        w3 = t.reshape(_CBLK // _PACK, _PACK, 32)
        w = jnp.concatenate(
            [w3[:, m, :] for m in range(_PACK)], axis=1)
        dst[...] = lax.bitcast_convert_type(w, jnp.float32)


_tc_pack = pl.pallas_call(
    _tc_pack_body,
    grid=(_GRID,),
    in_specs=[
        pl.BlockSpec((_D, _CBLK), lambda i: (0, i)),
        pl.BlockSpec((_D, _CBLK), lambda i: (0, i)),
    ],
    out_specs=[
        pl.BlockSpec((_CBLK // _PACK, _DP), lambda i: (i, 0)),
        pl.BlockSpec((_CBLK // _PACK, _DP), lambda i: (i, 0)),
    ],
    out_shape=[
        jax.ShapeDtypeStruct((_VP, _DP), jnp.float32),
        jax.ShapeDtypeStruct((_VP, _DP), jnp.float32),
    ],
)


def kernel(inputs, user_table, item_table, W, b):
    idx = inputs.astype(jnp.int32)
    iu = idx[:, 0].reshape(_NW * _NCHUNK, _CHUNK)
    iv = idx[:, 1].reshape(_NW * _NCHUNK, _CHUNK)
    up, vp = _tc_pack(jnp.transpose(user_table), jnp.transpose(item_table))
    w = W.reshape(_D).astype(jnp.float32)
    wsplit = jnp.concatenate(
        [w[0:16], w[32:48], w[16:32], w[48:64]])
    bias = jnp.broadcast_to(b.astype(jnp.float32), (_L,))
    out = _gmf_sc(up, vp, iu, iv, wsplit, bias)
    return out.reshape(_B, 1)


# speedup vs baseline: 7.7841x; 1.7520x over previous
import functools

import jax
import jax.numpy as jnp
from jax import lax
from jax.experimental import pallas as pl
from jax.experimental.pallas import tpu as pltpu
from jax.experimental.pallas import tpu_sc as plsc

_B = 16384
_D = 64
_L = 16
_NC = 2
_NS = 16
_NW = _NC * _NS
_BPW = _B // _NW
_PACK = 4
_DP = 128
_DW = _D // 2
_CHUNK = 128
_NCHUNK = _BPW // _CHUNK


@functools.partial(
    pl.kernel,
    mesh=plsc.VectorSubcoreMesh(core_axis_name="c", subcore_axis_name="s"),
    out_type=jax.ShapeDtypeStruct((_B,), jnp.float32),
    compiler_params=pltpu.CompilerParams(
        use_tc_tiling_on_sc=True, needs_layout_passes=False),
    scratch_types=[
        pltpu.VMEM((_NCHUNK, _CHUNK), jnp.int32),
        pltpu.VMEM((_NCHUNK, _CHUNK), jnp.int32),
        pltpu.VMEM((_NCHUNK, _CHUNK), jnp.int32),
        pltpu.VMEM((_NCHUNK, _CHUNK), jnp.int32),
        pltpu.VMEM((_CHUNK, _DP), jnp.float32),
        pltpu.VMEM((_CHUNK, _DP), jnp.float32),
        pltpu.VMEM((_CHUNK, _DP), jnp.float32),
        pltpu.VMEM((_CHUNK, _DP), jnp.float32),
        pltpu.VMEM((_D,), jnp.float32),
        pltpu.VMEM((_L,), jnp.float32),
        pltpu.VMEM((_BPW,), jnp.float32),
        pltpu.SemaphoreType.DMA,
        pltpu.SemaphoreType.DMA,
    ],
)
def _gmf_sc(user_hbm, item_hbm, iu_hbm, iv_hbm, w_hbm, bias_hbm, out_hbm,
            iu_v, iv_v, iu4_v, iv4_v, ub0, ub1, vb0, vb1, w_v, bias_v, out_v,
            sem0, sem1):
    wid = lax.axis_index("s") * _NC + lax.axis_index("c")
    base = wid * _BPW
    ubufs = (ub0, ub1)
    vbufs = (vb0, vb1)
    sems = (sem0, sem1)

    pltpu.sync_copy(iu_hbm.at[pl.ds(wid * _NCHUNK, _NCHUNK)], iu_v)
    pltpu.sync_copy(iv_hbm.at[pl.ds(wid * _NCHUNK, _NCHUNK)], iv_v)
    pltpu.sync_copy(w_hbm, w_v)
    pltpu.sync_copy(bias_hbm, bias_v)

    for j in range(_NCHUNK):
        for t in range(_CHUNK // _L):
            s = pl.ds(t * _L, _L)
            iu = iu_v[j, s]
            iv = iv_v[j, s]
            iu4_v[j, s] = (lax.shift_right_logical(iu, 9) * 128
                           + jnp.bitwise_and(iu, 127))
            iv4_v[j, s] = (lax.shift_right_logical(iv, 9) * 128
                           + jnp.bitwise_and(iv, 127))

    def fire(j):
        k = j % 2
        return (pltpu.async_copy(user_hbm.at[iu4_v.at[j]], ubufs[k], sems[k]),
                pltpu.async_copy(item_hbm.at[iv4_v.at[j]], vbufs[k], sems[k]))

    w0 = w_v[pl.ds(0 * _L, _L)]
    w1 = w_v[pl.ds(1 * _L, _L)]
    w2 = w_v[pl.ds(2 * _L, _L)]
    w3 = w_v[pl.ds(3 * _L, _L)]

    lane = lax.iota(jnp.int32, _L)
    lo_half = lane < (_L // 2)
    perm_even = (lane * 2) % _L
    perm_odd = perm_even + 1

    def shuf(x, perm):
        return lax.gather(
            x, perm[:, None],
            lax.GatherDimensionNumbers(
                offset_dims=(), collapsed_slice_dims=(0,), start_index_map=(0,)),
            slice_sizes=(1,),
            mode=lax.GatherScatterMode.PROMISE_IN_BOUNDS)

    def hadd(a, b):
        return jnp.where(lo_half,
                         shuf(a, perm_even) + shuf(a, perm_odd),
                         shuf(b, perm_even) + shuf(b, perm_odd))

    def unpack2(x16):
        b32 = plsc.bitcast(x16, jnp.bfloat16)
        return plsc.unpack(b32, format=plsc.PackFormat.INTERLEAVED)

    cps = {0: fire(0)}
    for j in range(_NCHUNK):
        if j + 1 < _NCHUNK:
            cps[j + 1] = fire(j + 1)
        for c in cps.pop(j):
            c.wait()
        u_rows = ubufs[j % 2]
        v_rows = vbufs[j % 2]

        def block_body(blk, carry, u_rows=u_rows, v_rows=v_rows, j=j):
            base_r = blk * _L
            qu = jnp.bitwise_and(lax.shift_right_logical(
                iu_v[j, pl.ds(base_r, _L)], 7), _PACK - 1) * _DW
            qv = jnp.bitwise_and(lax.shift_right_logical(
                iv_v[j, pl.ds(base_r, _L)], 7), _PACK - 1) * _DW
            ps = []
            for k in range(_L):
                r = base_r + k
                bcast_k = lane * 0 + k
                qu_k = shuf(qu, bcast_k)
                qv_k = shuf(qv, bcast_k)
                row_id = lane * 0 + r
                pu0 = plsc.load_gather(u_rows, [row_id, qu_k + lane])
                pu1 = plsc.load_gather(u_rows, [row_id, qu_k + (lane + _L)])
                pv0 = plsc.load_gather(v_rows, [row_id, qv_k + lane])
                pv1 = plsc.load_gather(v_rows, [row_id, qv_k + (lane + _L)])
                ue0, uo0 = unpack2(pu0)
                ue1, uo1 = unpack2(pu1)
                ve0, vo0 = unpack2(pv0)
                ve1, vo1 = unpack2(pv1)
                p = (ue0 * w0) * ve0 + (uo0 * w1) * vo0
                p = p + (ue1 * w2) * ve1 + (uo1 * w3) * vo1
                ps.append(p)
            while len(ps) > 1:
                ps = [hadd(ps[i], ps[i + 1]) for i in range(0, len(ps), 2)]
            out_v[pl.ds(j * _CHUNK + base_r, _L)] = ps[0]
            return carry

        lax.fori_loop(0, _CHUNK // _L, block_body, 0)

    bv = bias_v[...]
    for i in range(_BPW // _L):
        x = out_v[pl.ds(i * _L, _L)] + bv
        out_v[pl.ds(i * _L, _L)] = 1.0 / (1.0 + jnp.exp(-x))

    pltpu.sync_copy(out_v, out_hbm.at[pl.ds(base, _BPW)])


_V = 100000
_VP = 25600
_CBLK = 4096
_GRID = (_V + _CBLK - 1) // _CBLK


def _tc_pack_body(ut_ref, vt_ref, up_ref, vp_ref):
    for src, dst in ((ut_ref, up_ref), (vt_ref, vp_ref)):
        x = lax.bitcast_convert_type(src[...], jnp.uint32)
        half = jnp.uint32(0x8000)
        hi_mask = jnp.uint32(0xFFFF0000)
        w = ((x[0:32, :] + half) >> 16) | ((x[32:64, :] + half) & hi_mask)
        for g in range(_CBLK // 512):
            c0 = g * 512
            s = jnp.concatenate(
                [w[:, c0 + 128 * p: c0 + 128 * (p + 1)] for p in range(4)],
                axis=0)
            t = jnp.transpose(s)
            dst[pl.ds(g * 128, 128), :] = lax.bitcast_convert_type(
                t, jnp.float32)


_tc_pack = pl.pallas_call(
    _tc_pack_body,
    grid=(_GRID,),
    in_specs=[
        pl.BlockSpec((_D, _CBLK), lambda i: (0, i)),
        pl.BlockSpec((_D, _CBLK), lambda i: (0, i)),
    ],
    out_specs=[
        pl.BlockSpec((_CBLK // _PACK, _DP), lambda i: (i, 0)),
        pl.BlockSpec((_CBLK // _PACK, _DP), lambda i: (i, 0)),
    ],
    out_shape=[
        jax.ShapeDtypeStruct((_VP, _DP), jnp.float32),
        jax.ShapeDtypeStruct((_VP, _DP), jnp.float32),
    ],
)


def kernel(inputs, user_table, item_table, W, b):
    idx = inputs.astype(jnp.int32)
    iu = idx[:, 0].reshape(_NW * _NCHUNK, _CHUNK)
    iv = idx[:, 1].reshape(_NW * _NCHUNK, _CHUNK)
    up, vp = _tc_pack(jnp.transpose(user_table), jnp.transpose(item_table))
    w = W.reshape(_D).astype(jnp.float32)
    wsplit = jnp.concatenate(
        [w[0:16], w[32:48], w[16:32], w[48:64]])
    bias = jnp.broadcast_to(b.astype(jnp.float32), (_L,))
    out = _gmf_sc(up, vp, iu, iv, wsplit, bias)
    return out.reshape(_B, 1)
